# baseline (speedup 1.0000x reference)
import functools

import jax
import jax.numpy as jnp
from jax import lax
from jax.experimental import pallas as pl
from jax.experimental.pallas import tpu as pltpu
from jax.experimental.pallas import tpu_sc as plsc

L = 64
N = 32
NHID = 1024
Q = 64
B = 16
TOPK = 8
BO = 8
NOCT = B // BO
LCH = 16
SCALE = 1.0 / 32.0

LANES = 16
QCH = Q // LANES
QH = 2


def _attn_kernel(k_ref, qt_ref, att_ref, smax_ref):
    l = pl.program_id(1)
    a = k_ref[:, 0].reshape(N * BO, LCH * NHID).astype(jnp.bfloat16)
    qt = qt_ref[...]
    parts = [
        jax.lax.dot_general(
            a[:, i * NHID:(i + 1) * NHID], qt, (((1,), (0,)), ((), ())),
            preferred_element_type=jnp.float32,
            precision=jax.lax.Precision.DEFAULT,
        )
        for i in range(LCH)
    ]
    s = parts[0]
    for p in parts[1:]:
        s = jnp.maximum(s, p)

    @pl.when(l == 0)
    def _():
        smax_ref[...] = s

    @pl.when(l > 0)
    def _():
        smax_ref[...] = jnp.maximum(smax_ref[...], s)

    @pl.when(l == pl.num_programs(1) - 1)
    def _():
        sm3 = smax_ref[...].reshape(N, BO, BO * Q)
        atts = []
        for bo in range(BO):
            logits = sm3[:, bo, bo * Q:(bo + 1) * Q] * SCALE
            m = jnp.max(logits, axis=0, keepdims=True)
            e = jnp.exp(logits - m)
            atts.append(e / jnp.sum(e, axis=0, keepdims=True))
        att_ref[...] = jnp.stack(atts, axis=0)


@functools.partial(
    pl.kernel,
    mesh=plsc.VectorSubcoreMesh(core_axis_name="c", subcore_axis_name="s"),
    out_type=jax.ShapeDtypeStruct((B * Q * TOPK,), jnp.int32),
    scratch_types=[
        pltpu.VMEM((N * Q,), jnp.float32),
        pltpu.VMEM((Q // QH * TOPK,), jnp.int32),
    ],
)
def _topk_sc(att_hbm, out_hbm, slab_v, idx_v):
    nc = 2
    wid = lax.axis_index("s") * nc + lax.axis_index("c")
    b = wid // QH
    qh = wid % QH
    pltpu.sync_copy(att_hbm.at[pl.ds(b * N * Q, N * Q)], slab_v)
    neg = jnp.full((LANES,), -jnp.inf, jnp.float32)
    for c in range(QCH // QH):
        qoff = (qh * (QCH // QH) + c) * LANES
        vals = [slab_v[pl.ds(n * Q + qoff, LANES)] for n in range(N)]
        for k in range(TOPK):
            m = vals[0]
            for n in range(1, N):
                m = jnp.maximum(m, vals[n])
            idx = jnp.full((LANES,), N, jnp.int32)
            for n in range(N - 1, -1, -1):
                idx = jnp.where(vals[n] >= m, n, idx)
            idx_v[pl.ds((c * TOPK + k) * LANES, LANES)] = idx
            for n in range(N):
                vals[n] = jnp.where(idx == n, neg, vals[n])
    pltpu.sync_copy(idx_v, out_hbm.at[pl.ds(wid * (Q // QH * TOPK), Q // QH * TOPK)])


def kernel(query, keys):
    k4 = keys.reshape(N, NOCT, BO, L * NHID)
    qt = jnp.transpose(query, (1, 2, 0)).reshape(NHID, B * Q)
    qt = qt.astype(jnp.bfloat16)
    att_bnq = pl.pallas_call(
        _attn_kernel,
        grid=(NOCT, L // LCH),
        in_specs=[
            pl.BlockSpec((N, 1, BO, LCH * NHID), lambda o, l: (0, o, 0, l)),
            pl.BlockSpec((NHID, BO * Q), lambda o, l: (0, o)),
        ],
        out_specs=pl.BlockSpec((BO, N, Q), lambda o, l: (o, 0, 0)),
        out_shape=jax.ShapeDtypeStruct((B, N, Q), jnp.float32),
        scratch_shapes=[pltpu.VMEM((N * BO, BO * Q), jnp.float32)],
    )(k4, qt)
    idx_flat = _topk_sc(att_bnq.reshape(B * N * Q))
    attention = jnp.transpose(att_bnq, (2, 0, 1))
    idx = idx_flat.reshape(B, QH, QCH // QH, TOPK, LANES)
    topk_indices = jnp.transpose(idx, (3, 1, 2, 4, 0)).reshape(TOPK, Q, B)
    return (attention, topk_indices)


# speedup vs baseline: 1.3148x; 1.3148x over previous
import jax
import jax.numpy as jnp
from jax.experimental import pallas as pl
from jax.experimental.pallas import tpu as pltpu

L = 64
N = 32
NHID = 1024
Q = 64
B = 16
TOPK = 8
BO = 8
NOCT = B // BO
LCH = 16
SCALE = 1.0 / 32.0


def _attn_kernel(k_ref, qt_ref, att_ref, idx_ref, smax_ref):
    l = pl.program_id(1)
    a = k_ref[:, 0].reshape(N * BO, LCH * NHID).astype(jnp.bfloat16)
    qt = qt_ref[...]
    parts = [
        jax.lax.dot_general(
            a[:, i * NHID:(i + 1) * NHID], qt, (((1,), (0,)), ((), ())),
            preferred_element_type=jnp.float32,
            precision=jax.lax.Precision.DEFAULT,
        )
        for i in range(LCH)
    ]
    s = parts[0]
    for p in parts[1:]:
        s = jnp.maximum(s, p)

    @pl.when(l == 0)
    def _():
        smax_ref[...] = s

    @pl.when(l > 0)
    def _():
        smax_ref[...] = jnp.maximum(smax_ref[...], s)

    @pl.when(l == pl.num_programs(1) - 1)
    def _():
        sm3 = smax_ref[...].reshape(N, BO, BO * Q)
        atts, idxs = [], []
        iota = jax.lax.broadcasted_iota(jnp.int32, (N, Q), 0)
        for bo in range(BO):
            logits = sm3[:, bo, bo * Q:(bo + 1) * Q] * SCALE
            m = jnp.max(logits, axis=0, keepdims=True)
            e = jnp.exp(logits - m)
            att = e / jnp.sum(e, axis=0, keepdims=True)
            atts.append(att)
            vals = att
            rows = []
            for _ in range(TOPK):
                cur = jnp.max(vals, axis=0, keepdims=True)
                idx = jnp.min(jnp.where(vals >= cur, iota, N), axis=0)
                rows.append(idx)
                vals = jnp.where(iota == idx[None, :], -jnp.inf, vals)
            idxs.append(jnp.stack(rows, axis=0))
        att_ref[...] = jnp.stack(atts, axis=0)
        idx_ref[...] = jnp.stack(idxs, axis=0)


def kernel(query, keys):
    k4 = keys.reshape(N, NOCT, BO, L * NHID)
    qt = jnp.transpose(query, (1, 2, 0)).reshape(NHID, B * Q)
    qt = qt.astype(jnp.bfloat16)
    att_bnq, idx_bkq = pl.pallas_call(
        _attn_kernel,
        grid=(NOCT, L // LCH),
        in_specs=[
            pl.BlockSpec((N, 1, BO, LCH * NHID), lambda o, l: (0, o, 0, l)),
            pl.BlockSpec((NHID, BO * Q), lambda o, l: (0, o)),
        ],
        out_specs=[
            pl.BlockSpec((BO, N, Q), lambda o, l: (o, 0, 0)),
            pl.BlockSpec((BO, TOPK, Q), lambda o, l: (o, 0, 0)),
        ],
        out_shape=[
            jax.ShapeDtypeStruct((B, N, Q), jnp.float32),
            jax.ShapeDtypeStruct((B, TOPK, Q), jnp.int32),
        ],
        scratch_shapes=[pltpu.VMEM((N * BO, BO * Q), jnp.float32)],
    )(k4, qt)
    attention = jnp.transpose(att_bnq, (2, 0, 1))
    topk_indices = jnp.transpose(idx_bkq, (1, 2, 0))
    return (attention, topk_indices)
